# baseline (speedup 1.0000x reference)
import jax
import jax.numpy as jnp
from jax import lax
from jax.experimental import pallas as pl
from jax.experimental.pallas import tpu as pltpu
from jax.experimental.pallas import tpu_sc as plsc

_SHAPE = (64, 8192)
_TABLE = _SHAPE[1]

_info = plsc.get_sparse_core_info()
_NC, _NS, _L = _info.num_cores, _info.num_subcores, _info.num_lanes
_NW = _NC * _NS
_BR, _BC = 8, 2048
_CG = _SHAPE[1] // _BC
_RVECS = _BC // _L


_HC = _BC // 2
_HV = _HC // _L


def _gather_body(x_hbm, idx_hbm, out_hbm, table_v, idx_v, out_v,
                 sem_t, sem_a, sem_b):
    wid = lax.axis_index("s") * _NC + lax.axis_index("c")
    r0 = (wid // _CG) * _BR
    c0 = (wid % _CG) * _BC
    table_cp = pltpu.async_copy(x_hbm, table_v, sem_t)
    idx_lo = pltpu.async_copy(
        idx_hbm.at[pl.ds(r0, _BR), pl.ds(c0, _HC)],
        idx_v.at[:, pl.ds(0, _HC)], sem_a)
    idx_hi = pltpu.async_copy(
        idx_hbm.at[pl.ds(r0, _BR), pl.ds(c0 + _HC, _HC)],
        idx_v.at[:, pl.ds(_HC, _HC)], sem_b)
    table_cp.wait()
    idx_lo.wait()

    @plsc.parallel_loop(0, _BR * _HV, unroll=8)
    def step_lo(i):
        r = i // _HV
        off = (i % _HV) * _L
        iv = idx_v[r, pl.ds(off, _L)]
        out_v[r, pl.ds(off, _L)] = plsc.load_gather(table_v, [iv])

    out_lo = pltpu.async_copy(
        out_v.at[:, pl.ds(0, _HC)],
        out_hbm.at[pl.ds(r0, _BR), pl.ds(c0, _HC)], sem_a)
    idx_hi.wait()

    @plsc.parallel_loop(0, _BR * _HV, unroll=8)
    def step_hi(i):
        r = i // _HV
        off = _HC + (i % _HV) * _L
        iv = idx_v[r, pl.ds(off, _L)]
        out_v[r, pl.ds(off, _L)] = plsc.load_gather(table_v, [iv])

    out_hi = pltpu.async_copy(
        out_v.at[:, pl.ds(_HC, _HC)],
        out_hbm.at[pl.ds(r0, _BR), pl.ds(c0 + _HC, _HC)], sem_b)
    out_lo.wait()
    out_hi.wait()


_gather = pl.kernel(
    _gather_body,
    out_type=jax.ShapeDtypeStruct(_SHAPE, jnp.float32),
    mesh=plsc.VectorSubcoreMesh(core_axis_name="c", subcore_axis_name="s"),
    scratch_types=[
        pltpu.VMEM((_TABLE,), jnp.float32),
        pltpu.VMEM((_BR, _BC), jnp.int32),
        pltpu.VMEM((_BR, _BC), jnp.float32),
        pltpu.SemaphoreType.DMA,
        pltpu.SemaphoreType.DMA,
        pltpu.SemaphoreType.DMA,
    ],
    compiler_params=pltpu.CompilerParams(
        needs_layout_passes=False, use_tc_tiling_on_sc=True),
)


def kernel(x, idx_tensor):
    return _gather(x, idx_tensor.astype(jnp.int32))


# speedup vs baseline: 1.0230x; 1.0230x over previous
import jax
import jax.numpy as jnp
from jax import lax
from jax.experimental import pallas as pl
from jax.experimental.pallas import tpu as pltpu
from jax.experimental.pallas import tpu_sc as plsc

_SHAPE = (64, 8192)
_TABLE = _SHAPE[1]

_info = plsc.get_sparse_core_info()
_NC, _NS, _L = _info.num_cores, _info.num_subcores, _info.num_lanes
_NW = _NC * _NS
_BR, _BC = 8, 2048
_CG = _SHAPE[1] // _BC
_RVECS = _BC // _L


def _gather_body(x_hbm, idx_hbm, out_hbm, table_v, idx_v, out_v, sem):
    wid = lax.axis_index("s") * _NC + lax.axis_index("c")
    r0 = (wid // _CG) * _BR
    c0 = (wid % _CG) * _BC
    table_cp = pltpu.async_copy(x_hbm, table_v, sem)
    idx_cp = pltpu.async_copy(
        idx_hbm.at[pl.ds(r0, _BR), pl.ds(c0, _BC)], idx_v, sem)
    table_cp.wait()
    idx_cp.wait()
    _HC = _BC // 2
    _HV = _HC // _L

    @plsc.parallel_loop(0, _BR * _HV, unroll=8)
    def step_lo(i):
        r = i // _HV
        off = (i % _HV) * _L
        iv = idx_v[r, pl.ds(off, _L)]
        out_v[r, pl.ds(off, _L)] = plsc.load_gather(table_v, [iv])

    out_lo = pltpu.async_copy(
        out_v.at[:, pl.ds(0, _HC)],
        out_hbm.at[pl.ds(r0, _BR), pl.ds(c0, _HC)], sem)

    @plsc.parallel_loop(0, _BR * _HV, unroll=8)
    def step_hi(i):
        r = i // _HV
        off = _HC + (i % _HV) * _L
        iv = idx_v[r, pl.ds(off, _L)]
        out_v[r, pl.ds(off, _L)] = plsc.load_gather(table_v, [iv])

    out_hi = pltpu.async_copy(
        out_v.at[:, pl.ds(_HC, _HC)],
        out_hbm.at[pl.ds(r0, _BR), pl.ds(c0 + _HC, _HC)], sem)
    out_lo.wait()
    out_hi.wait()


_gather = pl.kernel(
    _gather_body,
    out_type=jax.ShapeDtypeStruct(_SHAPE, jnp.float32),
    mesh=plsc.VectorSubcoreMesh(core_axis_name="c", subcore_axis_name="s"),
    scratch_types=[
        pltpu.VMEM((_TABLE,), jnp.float32),
        pltpu.VMEM((_BR, _BC), jnp.int32),
        pltpu.VMEM((_BR, _BC), jnp.float32),
        pltpu.SemaphoreType.DMA,
    ],
    compiler_params=pltpu.CompilerParams(
        needs_layout_passes=False, use_tc_tiling_on_sc=True),
)


def kernel(x, idx_tensor):
    return _gather(x, idx_tensor.astype(jnp.int32))
